# Initial kernel scaffold; baseline (speedup 1.0000x reference)
import jax
import jax.numpy as jnp
from jax.experimental import pallas as pl


def kernel(weights, query_distances, query_quantities, query_bid_size, reference_distances, reference_quantities, reference_bid_size, reference_prices):
    raise NotImplementedError("write your pallas kernel here")


# speedup vs baseline: 4.0251x; 4.0251x over previous
import functools

import jax
import jax.numpy as jnp
from jax import lax
from jax.experimental import pallas as pl
from jax.experimental.pallas import tpu as pltpu
from jax.experimental.pallas import tpu_sc as plsc

N = 100000
TOPK = 64
LANES = 128
NPAD = 100352
ROWS = NPAD // LANES
NW = 16
CH = NPAD // NW
NV = CH // 16
SLOT = 256
NEG = float("-inf")


def _sim_body(scal_ref, rd_ref, rq_ref, rb_ref, out_ref):
    w0 = scal_ref[0]
    w1 = scal_ref[1]
    w2 = scal_ref[2]
    qd = scal_ref[3]
    qq = scal_ref[4]
    qb = scal_ref[5]
    s0 = w0 * (-jnp.abs(qd - rd_ref[...]))
    s1 = w1 * (-jnp.abs(qq - rq_ref[...]))
    s2 = w2 * (-jnp.abs(qb - rb_ref[...]))
    m = jnp.maximum(jnp.maximum(s0, s1), s2)
    e0 = jnp.exp(s0 - m)
    e1 = jnp.exp(s1 - m)
    e2 = jnp.exp(s2 - m)
    t = (e0 + e2) + e1
    p0 = e0 / t
    p1 = e1 / t
    p2 = e2 / t
    s = ((p0 + p2) + p1) / 3.0
    row = lax.broadcasted_iota(jnp.int32, (ROWS, LANES), 0)
    col = lax.broadcasted_iota(jnp.int32, (ROWS, LANES), 1)
    flat = row * LANES + col
    out_ref[...] = jnp.where(flat < N, s, NEG)


def _similarities(scal, rd2, rq2, rb2):
    return pl.pallas_call(
        _sim_body,
        in_specs=[pl.BlockSpec(memory_space=pltpu.SMEM),
                  pl.BlockSpec(memory_space=pltpu.VMEM),
                  pl.BlockSpec(memory_space=pltpu.VMEM),
                  pl.BlockSpec(memory_space=pltpu.VMEM)],
        out_specs=pl.BlockSpec(memory_space=pltpu.VMEM),
        out_shape=jax.ShapeDtypeStruct((ROWS, LANES), jnp.float32),
    )(scal, rd2, rq2, rb2)


_mesh = plsc.VectorSubcoreMesh(core_axis_name="c", subcore_axis_name="s",
                               num_cores=1)


@functools.partial(
    pl.kernel,
    out_type=(jax.ShapeDtypeStruct((16,), jnp.float32),
              jax.ShapeDtypeStruct((NW * SLOT,), jnp.float32)),
    mesh=_mesh,
    compiler_params=pltpu.CompilerParams(needs_layout_passes=False),
    scratch_types=[
        pltpu.VMEM((CH,), jnp.float32),
        pltpu.VMEM((CH,), jnp.float32),
        pltpu.VMEM((TOPK,), jnp.float32),
        pltpu.VMEM((TOPK,), jnp.float32),
        pltpu.VMEM((SLOT,), jnp.float32),
        pltpu.VMEM((NW * SLOT,), jnp.float32),
        pltpu.VMEM((16,), jnp.float32),
    ],
)
def _sc_topk(s_hbm, p_hbm, res_hbm, cand_hbm, sv, pv, selv, selp, stage,
             mbuf, obuf):
    wid = lax.axis_index("s")
    base = wid * CH
    pltpu.sync_copy(s_hbm.at[pl.ds(base, CH)], sv)
    pltpu.sync_copy(p_hbm.at[pl.ds(base, CH)], pv)
    lanes = lax.iota(jnp.int32, 16)

    def mx_body(j, acc):
        return jnp.maximum(acc, sv[pl.ds(j * 16, 16)])
    accm = lax.fori_loop(0, NV, mx_body, jnp.full((16,), NEG, jnp.float32))
    g0 = jnp.max(accm)

    def lvl_pass(g):
        def body(j, carry):
            cnt, nmax = carry
            x = sv[pl.ds(j * 16, 16)]
            cnt = cnt + (x == g).astype(jnp.int32)
            nmax = jnp.maximum(nmax, jnp.where(x < g, x, NEG))
            return cnt, nmax
        cnt, nmax = lax.fori_loop(
            0, NV, body,
            (jnp.zeros((16,), jnp.int32), jnp.full((16,), NEG, jnp.float32)))
        return jnp.sum(cnt), jnp.max(nmax)

    def w_cond(st):
        return st[0]

    def w_body(st):
        _, g, cum = st
        c_w, m2_w = lvl_pass(g)
        done = (cum + c_w) >= TOPK
        g_next = jnp.where(done, g, m2_w)
        cum_next = jnp.where(done, cum, cum + c_w)
        return (~done, g_next, cum_next)

    _, g_fin, cum_prev = lax.while_loop(
        w_cond, w_body, (jnp.bool_(True), g0, jnp.int32(0)))
    quota = TOPK - cum_prev

    def sel_body(j, carry):
        cursor, rem = carry
        x = sv[pl.ds(j * 16, 16)]
        pr = pv[pl.ds(j * 16, 16)]
        gt = x > g_fin
        eq = x == g_fin
        eqrank = plsc.cumsum(eq.astype(jnp.int32))
        eq_take = eq & (eqrank <= rem)
        take = gt | eq_take
        tvec = take.astype(jnp.int32)
        pos = cursor + plsc.cumsum(tvec) - 1
        plsc.store_scatter(selv, [pos], x, mask=take)
        plsc.store_scatter(selp, [pos], pr, mask=take)
        return cursor + jnp.sum(tvec), rem - jnp.sum(eq_take.astype(jnp.int32))

    lax.fori_loop(0, NV, sel_body, (jnp.int32(0), quota))

    for k in range(TOPK // 16):
        stage[pl.ds(k * 16, 16)] = selv[pl.ds(k * 16, 16)]
        stage[pl.ds(TOPK + k * 16, 16)] = selp[pl.ds(k * 16, 16)]
    for k in range(2 * TOPK, SLOT, 16):
        stage[pl.ds(k, 16)] = jnp.zeros((16,), jnp.float32)
    pltpu.sync_copy(stage, cand_hbm.at[pl.ds(wid * SLOT, SLOT)])
    plsc.subcore_barrier()

    @pl.when(wid == 0)
    def _():
        pltpu.sync_copy(cand_hbm, mbuf)

        def cvec(w, k):
            return mbuf[pl.ds(w * SLOT + k * 16, 16)]

        def cprice(w, k):
            return mbuf[pl.ds(w * SLOT + TOPK + k * 16, 16)]

        gacc = jnp.full((16,), NEG, jnp.float32)
        for w in range(NW):
            for k in range(TOPK // 16):
                gacc = jnp.maximum(gacc, cvec(w, k))
        gg0 = jnp.max(gacc)

        def glvl(g):
            cnt = jnp.zeros((16,), jnp.int32)
            nmax = jnp.full((16,), NEG, jnp.float32)
            for w in range(NW):
                for k in range(TOPK // 16):
                    x = cvec(w, k)
                    cnt = cnt + (x == g).astype(jnp.int32)
                    nmax = jnp.maximum(nmax, jnp.where(x < g, x, NEG))
            return jnp.sum(cnt), jnp.max(nmax)

        def gw_body(st):
            _, g, cum = st
            c_w, m2_w = glvl(g)
            done = (cum + c_w) >= TOPK
            return (~done, jnp.where(done, g, m2_w),
                    jnp.where(done, cum, cum + c_w))

        _, gg_fin, gcum = lax.while_loop(
            w_cond, gw_body, (jnp.bool_(True), gg0, jnp.int32(0)))
        need = TOPK - gcum

        accw = jnp.zeros((16,), jnp.float32)
        accpw = jnp.zeros((16,), jnp.float32)
        prefix = jnp.int32(0)
        for w in range(NW):
            eqs = []
            ec = jnp.int32(0)
            for k in range(TOPK // 16):
                e = cvec(w, k) == gg_fin
                eqs.append(e)
                ec = ec + jnp.sum(e.astype(jnp.int32))
            quota_w = jnp.clip(need - prefix, 0, ec)
            prefix = prefix + ec
            roff = jnp.int32(0)
            for k in range(TOPK // 16):
                x = cvec(w, k)
                p = cprice(w, k)
                e = eqs[k]
                gt = x > gg_fin
                eqr = plsc.cumsum(e.astype(jnp.int32)) + roff
                roff = roff + jnp.sum(e.astype(jnp.int32))
                take = gt | (e & (eqr <= quota_w))
                wgt = 1.0 / ((1.0 - x) + 1e-6)
                accw = accw + jnp.where(take, wgt, 0.0)
                accpw = accpw + jnp.where(take, wgt * p, 0.0)

        obuf[...] = (jnp.full((16,), jnp.sum(accpw))
                     / jnp.full((16,), jnp.sum(accw)))
        pltpu.sync_copy(obuf, res_hbm)


def kernel(weights, query_distances, query_quantities, query_bid_size,
           reference_distances, reference_quantities, reference_bid_size,
           reference_prices):
    scal = jnp.concatenate([weights, query_distances, query_quantities,
                            query_bid_size])
    def pad2(x):
        return jnp.pad(x.reshape(-1), (0, NPAD - N)).reshape(ROWS, LANES)
    sims = _similarities(scal, pad2(reference_distances),
                         pad2(reference_quantities),
                         pad2(reference_bid_size))
    s_flat = sims.reshape(-1)
    p_flat = jnp.pad(reference_prices.reshape(-1), (0, NPAD - N))
    res, _ = _sc_topk(s_flat, p_flat)
    return res[0]
